# baseline (speedup 1.0000x reference)
import functools

import jax
import jax.numpy as jnp
from jax import lax
from jax.experimental import pallas as pl
from jax.experimental.pallas import tpu as pltpu
from jax.experimental.pallas import tpu_sc as plsc

VOCAB = 1000000
DIM = 64
BATCH = 16384

NUM_CORES = 2
NUM_SUBCORES = 16
NW = NUM_CORES * NUM_SUBCORES
B_PER_W = BATCH // NW
CHUNK = 128
NCHUNK = B_PER_W // CHUNK
LANES = 16

_mesh = plsc.VectorSubcoreMesh(core_axis_name="c", subcore_axis_name="s")


@functools.partial(
    pl.kernel,
    out_type=jax.ShapeDtypeStruct((BATCH, 2 * DIM), jnp.float32),
    mesh=_mesh,
    scratch_types=[
        pltpu.VMEM((NCHUNK, CHUNK), jnp.int32),
        pltpu.VMEM((NCHUNK, CHUNK), jnp.int32),
        pltpu.VMEM((B_PER_W, 2 * DIM), jnp.float32),
        pltpu.SemaphoreType.DMA,
    ],
    compiler_params=pltpu.CompilerParams(use_tc_tiling_on_sc=True),
)
def _sc_gather(idx_hbm, table_hbm, out_hbm, idx_v, row_v, rows_v, sem):
    wid = lax.axis_index("s") * NUM_CORES + lax.axis_index("c")
    base = wid * B_PER_W
    pltpu.sync_copy(idx_hbm.at[wid], idx_v)
    for c in range(NCHUNK):
        for j in range(CHUNK // LANES):
            v = idx_v[c, pl.ds(j * LANES, LANES)]
            row_v[c, pl.ds(j * LANES, LANES)] = jax.lax.shift_right_logical(v, 1)
    copies = []
    for c in range(NCHUNK):
        copies.append(
            pltpu.async_copy(
                table_hbm.at[row_v.at[c]],
                rows_v.at[pl.ds(c * CHUNK, CHUNK)],
                sem,
            )
        )
    for cp in copies:
        cp.wait()
    pltpu.sync_copy(rows_v, out_hbm.at[pl.ds(base, B_PER_W)])


_TC_ROWS = 1024


def _tc_select_body(idx_ref, pairs_ref, out_ref):
    odd = (idx_ref[...] & 1).astype(jnp.bool_)
    x = pairs_ref[...]
    out_ref[...] = jnp.where(odd, x[:, DIM:], x[:, :DIM])


_tc_select = pl.pallas_call(
    _tc_select_body,
    grid=(BATCH // _TC_ROWS,),
    in_specs=[
        pl.BlockSpec((_TC_ROWS, 1), lambda i: (i, 0)),
        pl.BlockSpec((_TC_ROWS, 2 * DIM), lambda i: (i, 0)),
    ],
    out_specs=pl.BlockSpec((_TC_ROWS, DIM), lambda i: (i, 0)),
    out_shape=jax.ShapeDtypeStruct((BATCH, DIM), jnp.float32),
)


def kernel(inputs, train_labels, embeddings):
    del train_labels
    table2 = embeddings.reshape(VOCAB // 2, 2 * DIM)
    idx3 = inputs.reshape(NW, NCHUNK, CHUNK)
    pairs = _sc_gather(idx3, table2)
    idx_tc = inputs.reshape(BATCH, 1)
    return _tc_select(idx_tc, pairs)
